# baseline (speedup 1.0000x reference)
import jax
import jax.numpy as jnp
from jax import lax
from jax.experimental import pallas as pl
from jax.experimental.pallas import tpu as pltpu
from jax.experimental.pallas import tpu_sc as plsc

VOCAB = 100000
HID = 128
MAX2D = 1024
TYPES = 2
B, L = 1024, 200
N = B * L
EPS = 1e-12

NC, NS, LANES = 2, 16, 16
NW = NC * NS
TPW = N // NW
C = 128
NCHUNK = TPW // C
SP32 = HID // 32


def _rsqrt16(v):
    y = plsc.bitcast(v, jnp.int32)
    y = jnp.int32(0x5F3759DF) - (y >> 1)
    r = plsc.bitcast(y, jnp.float32)
    for _ in range(3):
        r = r * (jnp.float32(1.5) - jnp.float32(0.5) * v * r * r)
    return r


def _body(ids_hbm, bb_hbm, tti_hbm, word_hbm, x_hbm, y_hbm, h_hbm, w_hbm,
          tt_hbm, gamma_hbm, beta_hbm, out_hbm, *sc):
    bb_v = sc[0:2]
    ids_v = sc[2:4]
    tti_v = sc[4:6]
    c0_v = sc[6:8]
    c1_v = sc[8:10]
    c2_v = sc[10:12]
    c3_v = sc[12:14]
    hh_v = sc[14:16]
    ww_v = sc[16:18]
    acc_v = sc[18:20]
    g_v, b_v = sc[20], sc[21]
    sem_g = sc[22:24]
    sem_i = sc[24:26]

    wid = lax.axis_index("c") * NS + lax.axis_index("s")
    base0 = wid * TPW

    pltpu.sync_copy(gamma_hbm, g_v)
    pltpu.sync_copy(beta_hbm, b_v)

    iota = lax.iota(jnp.int32, LANES)
    g32 = [g_v[pl.ds(s * 32, 32)] for s in range(SP32)]
    b32 = [b_v[pl.ds(s * 32, 32)] for s in range(SP32)]
    zero32 = jnp.zeros((32,), jnp.bfloat16)

    def fire_idx(base, p):
        pltpu.async_copy(ids_hbm.at[pl.ds(base, C)], ids_v[p], sem_i[p])
        pltpu.async_copy(bb_hbm.at[pl.ds(base * 4, C * 4)], bb_v[p], sem_i[p])
        pltpu.async_copy(tti_hbm.at[pl.ds(base, C)], tti_v[p], sem_i[p])

    def wait_idx(base, p):
        pltpu.make_async_copy(ids_hbm.at[pl.ds(base, C)], ids_v[p],
                              sem_i[p]).wait()
        pltpu.make_async_copy(bb_hbm.at[pl.ds(base * 4, C * 4)], bb_v[p],
                              sem_i[p]).wait()
        pltpu.make_async_copy(tti_hbm.at[pl.ds(base, C)], tti_v[p],
                              sem_i[p]).wait()

    def stage_and_fire(base, p):
        wait_idx(base, p)
        for i in range(C // LANES):
            f16 = (iota + i * LANES) * 4
            c0 = plsc.load_gather(bb_v[p], [f16])
            c1 = plsc.load_gather(bb_v[p], [f16 + 1])
            c2 = plsc.load_gather(bb_v[p], [f16 + 2])
            c3 = plsc.load_gather(bb_v[p], [f16 + 3])
            sl = pl.ds(i * LANES, LANES)
            c0_v[p][sl] = c0
            c1_v[p][sl] = c1
            c2_v[p][sl] = c2
            c3_v[p][sl] = c3
            hh_v[p][sl] = c3 - c1
            ww_v[p][sl] = c2 - c0

        av = acc_v[p]

        def zero_body(t, _):
            for s in range(SP32):
                av[t, pl.ds(s * 32, 32)] = zero32
            return 0

        lax.fori_loop(0, C, zero_body, 0)

        pltpu.async_copy(word_hbm.at[ids_v[p]], av, sem_g[p], add=True)
        pltpu.async_copy(x_hbm.at[c0_v[p]], av, sem_g[p], add=True)
        pltpu.async_copy(y_hbm.at[c1_v[p]], av, sem_g[p], add=True)
        pltpu.async_copy(x_hbm.at[c2_v[p]], av, sem_g[p], add=True)
        pltpu.async_copy(y_hbm.at[c3_v[p]], av, sem_g[p], add=True)
        pltpu.async_copy(h_hbm.at[hh_v[p]], av, sem_g[p], add=True)
        pltpu.async_copy(w_hbm.at[ww_v[p]], av, sem_g[p], add=True)
        pltpu.async_copy(tt_hbm.at[tti_v[p]], av, sem_g[p], add=True)

    def drain_gathers(p):
        for _ in range(8):
            pltpu.make_async_copy(word_hbm.at[ids_v[p]], acc_v[p],
                                  sem_g[p]).wait()

    himask = jnp.full((LANES,), jnp.int32(-65536))

    def compute(base, p):
        av = acc_v[p]

        def tok_body(t, _):
            ssum = jnp.zeros((LANES,), jnp.float32)
            ssq = jnp.zeros((LANES,), jnp.float32)
            a32 = []
            for s in range(SP32):
                a = av[t, pl.ds(s * 32, 32)]
                a32.append(a)
                w = plsc.bitcast(a, jnp.int32)
                e = plsc.bitcast(w << 16, jnp.float32)
                o = plsc.bitcast(w & himask, jnp.float32)
                ssum = ssum + (e + o)
                ssq = ssq + e * e + o * o
            tot = jnp.sum(ssum)
            tot2 = jnp.sum(ssq)
            mean = tot * jnp.float32(1.0 / HID)
            var = tot2 * jnp.float32(1.0 / HID) - mean * mean
            rv = _rsqrt16(jnp.broadcast_to(var + jnp.float32(EPS), (LANES,)))
            mv = jnp.broadcast_to(mean, (LANES,))
            rv32 = plsc.pack(rv, rv, format=plsc.PackFormat.INTERLEAVED)
            m32 = plsc.pack(mv, mv, format=plsc.PackFormat.INTERLEAVED)
            for s in range(SP32):
                o32 = (a32[s] - m32) * (rv32 * g32[s]) + b32[s]
                av[t, pl.ds(s * 32, 32)] = o32
            return 0

        lax.fori_loop(0, C, tok_body, 0)
        pltpu.sync_copy(av, out_hbm.at[pl.ds(base, C)])

    fire_idx(base0, 0)
    stage_and_fire(base0, 0)
    fire_idx(base0 + C, 1)

    def pair_body(i, _):
        for b in (0, 1):
            ci = 2 * i + b
            base = base0 + ci * C

            @pl.when(ci + 1 < NCHUNK)
            def _():
                stage_and_fire(base + C, 1 - b)

            drain_gathers(b)

            @pl.when(ci + 2 < NCHUNK)
            def _():
                fire_idx(base + 2 * C, b)

            compute(base, b)
        return 0

    lax.fori_loop(0, NCHUNK // 2, pair_body, 0)


@jax.jit
def _run(ids, bb, tti, word_emb, x_pos, y_pos, h_pos, w_pos, tt_emb, gamma, beta):
    mesh = plsc.VectorSubcoreMesh(core_axis_name="c", subcore_axis_name="s")
    dbl = lambda t: [t, t]
    f = pl.kernel(
        _body,
        out_type=jax.ShapeDtypeStruct((N, HID), jnp.bfloat16),
        mesh=mesh,
        compiler_params=pltpu.CompilerParams(needs_layout_passes=False,
                                             use_tc_tiling_on_sc=False),
        scratch_types=(
            dbl(pltpu.VMEM((C * 4,), jnp.int32))
            + dbl(pltpu.VMEM((C,), jnp.int32))
            + dbl(pltpu.VMEM((C,), jnp.int32))
            + dbl(pltpu.VMEM((C,), jnp.int32))
            + dbl(pltpu.VMEM((C,), jnp.int32))
            + dbl(pltpu.VMEM((C,), jnp.int32))
            + dbl(pltpu.VMEM((C,), jnp.int32))
            + dbl(pltpu.VMEM((C,), jnp.int32))
            + dbl(pltpu.VMEM((C,), jnp.int32))
            + dbl(pltpu.VMEM((C, HID), jnp.bfloat16))
            + [pltpu.VMEM((HID,), jnp.bfloat16)]
            + [pltpu.VMEM((HID,), jnp.bfloat16)]
            + dbl(pltpu.SemaphoreType.DMA)
            + dbl(pltpu.SemaphoreType.DMA)
        ),
    )
    return f(ids, bb, tti, word_emb, x_pos, y_pos, h_pos, w_pos, tt_emb, gamma, beta)


def kernel(input_ids, bbox, token_type_ids, word_emb, x_pos, y_pos, h_pos, w_pos,
           tt_emb, gamma, beta):
    ids = input_ids.reshape(-1).astype(jnp.int32)
    bb = bbox.reshape(-1).astype(jnp.int32)
    tti = token_type_ids.reshape(-1).astype(jnp.int32)
    cast = lambda t: t.astype(jnp.bfloat16)
    out = _run(ids, bb, tti, cast(word_emb), cast(x_pos), cast(y_pos),
               cast(h_pos), cast(w_pos), cast(tt_emb), cast(gamma), cast(beta))
    return out.astype(jnp.float32).reshape(input_ids.shape + (HID,))


# speedup vs baseline: 7.7179x; 7.7179x over previous
import jax
import jax.numpy as jnp
from jax import lax
from jax.experimental import pallas as pl
from jax.experimental.pallas import tpu as pltpu
from jax.experimental.pallas import tpu_sc as plsc

VOCAB = 100000
HID = 128
MAX2D = 1024
TYPES = 2
B, L = 1024, 200
N = B * L
EPS = 1e-12

NC, NS, LANES = 2, 16, 16
NW = NC * NS
TPW = N // NW
C = 128
NCHUNK = TPW // C
SPANS = HID // LANES


def _rsqrt16(v):
    y = plsc.bitcast(v, jnp.int32)
    y = jnp.int32(0x5F3759DF) - (y >> 1)
    r = plsc.bitcast(y, jnp.float32)
    for _ in range(3):
        r = r * (jnp.float32(1.5) - jnp.float32(0.5) * v * r * r)
    return r


def _body(ids_hbm, bb_hbm, tti_hbm, word_hbm, x_hbm, y_hbm, h_hbm, w_hbm,
          tt_hbm, gamma_hbm, beta_hbm, out_hbm, *sc):
    bb_v = sc[0:2]
    ids_v = sc[2:4]
    tti_v = sc[4:6]
    c0_v = sc[6:8]
    c1_v = sc[8:10]
    c2_v = sc[10:12]
    c3_v = sc[12:14]
    hh_v = sc[14:16]
    ww_v = sc[16:18]
    acc_v = sc[18:20]
    g_v, b_v, tt_v = sc[20], sc[21], sc[22]
    sem_g = sc[23:25]
    sem_i = sc[25:27]
    sem_o = sc[27:29]

    wid = lax.axis_index("c") * NS + lax.axis_index("s")
    base0 = wid * TPW

    pltpu.sync_copy(gamma_hbm, g_v)
    pltpu.sync_copy(beta_hbm, b_v)
    pltpu.sync_copy(tt_hbm, tt_v)

    iota = lax.iota(jnp.int32, LANES)
    gs = [g_v[pl.ds(s * LANES, LANES)] for s in range(SPANS)]
    bs = [b_v[pl.ds(s * LANES, LANES)] for s in range(SPANS)]
    tt0 = [tt_v[0, pl.ds(s * LANES, LANES)] for s in range(SPANS)]
    tt1 = [tt_v[1, pl.ds(s * LANES, LANES)] for s in range(SPANS)]

    def fire_idx(base, p):
        pltpu.async_copy(ids_hbm.at[pl.ds(base, C)], ids_v[p], sem_i[p])
        pltpu.async_copy(bb_hbm.at[pl.ds(base * 4, C * 4)], bb_v[p], sem_i[p])
        pltpu.async_copy(tti_hbm.at[pl.ds(base, C)], tti_v[p], sem_i[p])

    def wait_idx(base, p):
        pltpu.make_async_copy(ids_hbm.at[pl.ds(base, C)], ids_v[p],
                              sem_i[p]).wait()
        pltpu.make_async_copy(bb_hbm.at[pl.ds(base * 4, C * 4)], bb_v[p],
                              sem_i[p]).wait()
        pltpu.make_async_copy(tti_hbm.at[pl.ds(base, C)], tti_v[p],
                              sem_i[p]).wait()

    def wait_out(p):
        pltpu.make_async_copy(acc_v[p], out_hbm.at[pl.ds(base0, C)],
                              sem_o[p]).wait()

    def stage_and_fire(base, p):
        wait_idx(base, p)
        for i in range(C // LANES):
            f16 = (iota + i * LANES) * 4
            c0 = plsc.load_gather(bb_v[p], [f16])
            c1 = plsc.load_gather(bb_v[p], [f16 + 1])
            c2 = plsc.load_gather(bb_v[p], [f16 + 2])
            c3 = plsc.load_gather(bb_v[p], [f16 + 3])
            sl = pl.ds(i * LANES, LANES)
            c0_v[p][sl] = c0
            c1_v[p][sl] = c1
            c2_v[p][sl] = c2
            c3_v[p][sl] = c3
            hh_v[p][sl] = c3 - c1
            ww_v[p][sl] = c2 - c0

        av, ttv = acc_v[p], tti_v[p]

        @pl.when(base - base0 >= 2 * C)
        def _():
            wait_out(p)

        def pre_body(t, _):
            tsel = plsc.load_gather(ttv, [jnp.full((LANES,), 0, jnp.int32) + t]) > 0
            for s in range(SPANS):
                av[t, pl.ds(s * LANES, LANES)] = jnp.where(tsel, tt1[s], tt0[s])
            return 0

        lax.fori_loop(0, C, pre_body, 0)

        pltpu.async_copy(word_hbm.at[ids_v[p]], av, sem_g[p], add=True)
        pltpu.async_copy(x_hbm.at[c0_v[p]], av, sem_g[p], add=True)
        pltpu.async_copy(y_hbm.at[c1_v[p]], av, sem_g[p], add=True)
        pltpu.async_copy(x_hbm.at[c2_v[p]], av, sem_g[p], add=True)
        pltpu.async_copy(y_hbm.at[c3_v[p]], av, sem_g[p], add=True)
        pltpu.async_copy(h_hbm.at[hh_v[p]], av, sem_g[p], add=True)
        pltpu.async_copy(w_hbm.at[ww_v[p]], av, sem_g[p], add=True)

    def drain_gathers(p):
        for _ in range(7):
            pltpu.make_async_copy(word_hbm.at[ids_v[p]], acc_v[p],
                                  sem_g[p]).wait()

    def compute(base, p):
        av = acc_v[p]

        def tok_body(t, _):
            ssum = jnp.zeros((LANES,), jnp.float32)
            ssq = jnp.zeros((LANES,), jnp.float32)
            aa = []
            for s in range(SPANS):
                a = av[t, pl.ds(s * LANES, LANES)]
                aa.append(a)
                ssum = ssum + a
                ssq = ssq + a * a
            tot = jnp.sum(ssum)
            tot2 = jnp.sum(ssq)
            mean = tot * jnp.float32(1.0 / HID)
            var = tot2 * jnp.float32(1.0 / HID) - mean * mean
            rv = _rsqrt16(jnp.broadcast_to(var + jnp.float32(EPS), (LANES,)))
            mv = jnp.broadcast_to(mean, (LANES,))
            mr = mv * rv
            for s in range(SPANS):
                av[t, pl.ds(s * LANES, LANES)] = (aa[s] * rv - mr) * gs[s] + bs[s]
            return 0

        lax.fori_loop(0, C, tok_body, 0)
        pltpu.async_copy(av, out_hbm.at[pl.ds(base, C)], sem_o[p])

    fire_idx(base0, 0)
    stage_and_fire(base0, 0)
    fire_idx(base0 + C, 1)

    def pair_body(i, _):
        for b in (0, 1):
            ci = 2 * i + b
            base = base0 + ci * C

            @pl.when(ci + 1 < NCHUNK)
            def _():
                stage_and_fire(base + C, 1 - b)

            drain_gathers(b)

            @pl.when(ci + 2 < NCHUNK)
            def _():
                fire_idx(base + 2 * C, b)

            compute(base, b)
        return 0

    lax.fori_loop(0, NCHUNK // 2, pair_body, 0)

    wait_out(0)
    wait_out(1)


@jax.jit
def _run(ids, bb, tti, word_emb, x_pos, y_pos, h_pos, w_pos, tt_emb, gamma, beta):
    mesh = plsc.VectorSubcoreMesh(core_axis_name="c", subcore_axis_name="s")
    dbl = lambda t: [t, t]
    f = pl.kernel(
        _body,
        out_type=jax.ShapeDtypeStruct((N, HID), jnp.float32),
        mesh=mesh,
        compiler_params=pltpu.CompilerParams(needs_layout_passes=False),
        scratch_types=(
            dbl(pltpu.VMEM((C * 4,), jnp.int32))
            + dbl(pltpu.VMEM((C,), jnp.int32))
            + dbl(pltpu.VMEM((C,), jnp.int32))
            + dbl(pltpu.VMEM((C,), jnp.int32))
            + dbl(pltpu.VMEM((C,), jnp.int32))
            + dbl(pltpu.VMEM((C,), jnp.int32))
            + dbl(pltpu.VMEM((C,), jnp.int32))
            + dbl(pltpu.VMEM((C,), jnp.int32))
            + dbl(pltpu.VMEM((C,), jnp.int32))
            + dbl(pltpu.VMEM((C, HID), jnp.float32))
            + [pltpu.VMEM((HID,), jnp.float32)]
            + [pltpu.VMEM((HID,), jnp.float32)]
            + [pltpu.VMEM((TYPES, HID), jnp.float32)]
            + dbl(pltpu.SemaphoreType.DMA)
            + dbl(pltpu.SemaphoreType.DMA)
            + dbl(pltpu.SemaphoreType.DMA)
        ),
    )
    return f(ids, bb, tti, word_emb, x_pos, y_pos, h_pos, w_pos, tt_emb, gamma, beta)


def kernel(input_ids, bbox, token_type_ids, word_emb, x_pos, y_pos, h_pos, w_pos,
           tt_emb, gamma, beta):
    ids = input_ids.reshape(-1).astype(jnp.int32)
    bb = bbox.reshape(-1).astype(jnp.int32)
    tti = token_type_ids.reshape(-1).astype(jnp.int32)
    out = _run(ids, bb, tti, word_emb, x_pos, y_pos, h_pos, w_pos, tt_emb,
               gamma, beta)
    return out.reshape(input_ids.shape + (HID,))


# speedup vs baseline: 9.0609x; 1.1740x over previous
import jax
import jax.numpy as jnp
from jax import lax
from jax.experimental import pallas as pl
from jax.experimental.pallas import tpu as pltpu
from jax.experimental.pallas import tpu_sc as plsc

VOCAB = 100000
HID = 128
MAX2D = 1024
TYPES = 2
B, L = 1024, 200
N = B * L
EPS = 1e-12

NC, NS, LANES = 2, 16, 16
NW = NC * NS
TPW = N // NW
C = 128
NCHUNK = TPW // C
SPANS = HID // LANES


def _rsqrt16(v):
    y = plsc.bitcast(v, jnp.int32)
    y = jnp.int32(0x5F3759DF) - (y >> 1)
    r = plsc.bitcast(y, jnp.float32)
    for _ in range(3):
        r = r * (jnp.float32(1.5) - jnp.float32(0.5) * v * r * r)
    return r


def _body(ids_hbm, bb_hbm, tti_hbm, word_hbm, x_hbm, y_hbm, h_hbm, w_hbm,
          tt_hbm, gamma_hbm, beta_hbm, out_hbm, *sc):
    bb_v = sc[0:2]
    ids_v = sc[2:4]
    tti_v = sc[4:6]
    c0_v = sc[6:8]
    c1_v = sc[8:10]
    c2_v = sc[10:12]
    c3_v = sc[12:14]
    hh_v = sc[14:16]
    ww_v = sc[16:18]
    acc_v = sc[18:20]
    g_v, b_v, tt_v = sc[20], sc[21], sc[22]
    sem_g = sc[23:25]
    sem_i = sc[25:27]
    sem_o = sc[27:29]

    wid = lax.axis_index("c") * NS + lax.axis_index("s")
    base0 = wid * TPW

    pltpu.sync_copy(gamma_hbm, g_v)
    pltpu.sync_copy(beta_hbm, b_v)
    pltpu.sync_copy(tt_hbm, tt_v)

    iota = lax.iota(jnp.int32, LANES)
    gs = [g_v[pl.ds(s * LANES, LANES)] for s in range(SPANS)]
    bs = [b_v[pl.ds(s * LANES, LANES)] for s in range(SPANS)]
    tt0 = [tt_v[0, pl.ds(s * LANES, LANES)] for s in range(SPANS)]
    tt1 = [tt_v[1, pl.ds(s * LANES, LANES)] for s in range(SPANS)]

    def fire_idx(base, p):
        pltpu.async_copy(ids_hbm.at[pl.ds(base, C)], ids_v[p], sem_i[p])
        pltpu.async_copy(bb_hbm.at[pl.ds(base * 4, C * 4)], bb_v[p], sem_i[p])
        pltpu.async_copy(tti_hbm.at[pl.ds(base, C)], tti_v[p], sem_i[p])

    def wait_idx(base, p):
        pltpu.make_async_copy(ids_hbm.at[pl.ds(base, C)], ids_v[p],
                              sem_i[p]).wait()
        pltpu.make_async_copy(bb_hbm.at[pl.ds(base * 4, C * 4)], bb_v[p],
                              sem_i[p]).wait()
        pltpu.make_async_copy(tti_hbm.at[pl.ds(base, C)], tti_v[p],
                              sem_i[p]).wait()

    def wait_out(p):
        pltpu.make_async_copy(acc_v[p], out_hbm.at[pl.ds(base0, C)],
                              sem_o[p]).wait()

    def stage_and_fire(base, p):
        wait_idx(base, p)
        for i in range(C // LANES):
            f16 = (iota + i * LANES) * 4
            c0 = plsc.load_gather(bb_v[p], [f16])
            c1 = plsc.load_gather(bb_v[p], [f16 + 1])
            c2 = plsc.load_gather(bb_v[p], [f16 + 2])
            c3 = plsc.load_gather(bb_v[p], [f16 + 3])
            sl = pl.ds(i * LANES, LANES)
            c0_v[p][sl] = c0
            c1_v[p][sl] = c1
            c2_v[p][sl] = c2
            c3_v[p][sl] = c3
            hh_v[p][sl] = c3 - c1
            ww_v[p][sl] = c2 - c0

        av, ttv = acc_v[p], tti_v[p]

        @pl.when(base - base0 >= 2 * C)
        def _():
            wait_out(p)

        def pre_body(t, _):
            tsel = plsc.load_gather(ttv, [jnp.full((LANES,), 0, jnp.int32) + t]) > 0
            for s in range(SPANS):
                av[t, pl.ds(s * LANES, LANES)] = jnp.where(tsel, tt1[s], tt0[s])
            return 0

        lax.fori_loop(0, C, pre_body, 0)

        pltpu.async_copy(word_hbm.at[ids_v[p]], av, sem_g[p], add=True)
        pltpu.async_copy(x_hbm.at[c0_v[p]], av, sem_g[p], add=True)
        pltpu.async_copy(y_hbm.at[c1_v[p]], av, sem_g[p], add=True)
        pltpu.async_copy(x_hbm.at[c2_v[p]], av, sem_g[p], add=True)
        pltpu.async_copy(y_hbm.at[c3_v[p]], av, sem_g[p], add=True)
        pltpu.async_copy(h_hbm.at[hh_v[p]], av, sem_g[p], add=True)
        pltpu.async_copy(w_hbm.at[ww_v[p]], av, sem_g[p], add=True)

    def drain_gathers(p):
        for _ in range(7):
            pltpu.make_async_copy(word_hbm.at[ids_v[p]], acc_v[p],
                                  sem_g[p]).wait()

    def compute(base, p):
        av = acc_v[p]

        def tok_body(th, _):
            ts = (2 * th, 2 * th + 1)
            aa = [[], []]
            stats = [None, None]
            for k, t in enumerate(ts):
                ssum = jnp.zeros((LANES,), jnp.float32)
                ssq = jnp.zeros((LANES,), jnp.float32)
                for s in range(SPANS):
                    a = av[t, pl.ds(s * LANES, LANES)]
                    aa[k].append(a)
                    ssum = ssum + a
                    ssq = ssq + a * a
                stats[k] = (jnp.sum(ssum), jnp.sum(ssq))
            for k, t in enumerate(ts):
                tot, tot2 = stats[k]
                mean = tot * jnp.float32(1.0 / HID)
                var = tot2 * jnp.float32(1.0 / HID) - mean * mean
                rv = _rsqrt16(jnp.broadcast_to(var + jnp.float32(EPS), (LANES,)))
                mv = jnp.broadcast_to(mean, (LANES,))
                mr = mv * rv
                for s in range(SPANS):
                    av[t, pl.ds(s * LANES, LANES)] = (aa[k][s] * rv - mr) * gs[s] + bs[s]
            return 0

        lax.fori_loop(0, C // 2, tok_body, 0)
        pltpu.async_copy(av, out_hbm.at[pl.ds(base, C)], sem_o[p])

    fire_idx(base0, 0)
    stage_and_fire(base0, 0)
    fire_idx(base0 + C, 1)

    def pair_body(i, _):
        for b in (0, 1):
            ci = 2 * i + b
            base = base0 + ci * C

            @pl.when(ci + 1 < NCHUNK)
            def _():
                stage_and_fire(base + C, 1 - b)

            drain_gathers(b)

            @pl.when(ci + 2 < NCHUNK)
            def _():
                fire_idx(base + 2 * C, b)

            compute(base, b)
        return 0

    lax.fori_loop(0, NCHUNK // 2, pair_body, 0)

    wait_out(0)
    wait_out(1)


@jax.jit
def _run(ids, bb, tti, word_emb, x_pos, y_pos, h_pos, w_pos, tt_emb, gamma, beta):
    mesh = plsc.VectorSubcoreMesh(core_axis_name="c", subcore_axis_name="s")
    dbl = lambda t: [t, t]
    f = pl.kernel(
        _body,
        out_type=jax.ShapeDtypeStruct((N, HID), jnp.float32),
        mesh=mesh,
        compiler_params=pltpu.CompilerParams(needs_layout_passes=False),
        scratch_types=(
            dbl(pltpu.VMEM((C * 4,), jnp.int32))
            + dbl(pltpu.VMEM((C,), jnp.int32))
            + dbl(pltpu.VMEM((C,), jnp.int32))
            + dbl(pltpu.VMEM((C,), jnp.int32))
            + dbl(pltpu.VMEM((C,), jnp.int32))
            + dbl(pltpu.VMEM((C,), jnp.int32))
            + dbl(pltpu.VMEM((C,), jnp.int32))
            + dbl(pltpu.VMEM((C,), jnp.int32))
            + dbl(pltpu.VMEM((C,), jnp.int32))
            + dbl(pltpu.VMEM((C, HID), jnp.float32))
            + [pltpu.VMEM((HID,), jnp.float32)]
            + [pltpu.VMEM((HID,), jnp.float32)]
            + [pltpu.VMEM((TYPES, HID), jnp.float32)]
            + dbl(pltpu.SemaphoreType.DMA)
            + dbl(pltpu.SemaphoreType.DMA)
            + dbl(pltpu.SemaphoreType.DMA)
        ),
    )
    return f(ids, bb, tti, word_emb, x_pos, y_pos, h_pos, w_pos, tt_emb, gamma, beta)


def kernel(input_ids, bbox, token_type_ids, word_emb, x_pos, y_pos, h_pos, w_pos,
           tt_emb, gamma, beta):
    ids = input_ids.reshape(-1).astype(jnp.int32)
    bb = bbox.reshape(-1).astype(jnp.int32)
    tti = token_type_ids.reshape(-1).astype(jnp.int32)
    out = _run(ids, bb, tti, word_emb, x_pos, y_pos, h_pos, w_pos, tt_emb,
               gamma, beta)
    return out.reshape(input_ids.shape + (HID,))
